# baseline (speedup 1.0000x reference)
import functools

import jax
import jax.numpy as jnp
from jax import lax
from jax.experimental import pallas as pl
from jax.experimental.pallas import tpu as pltpu
from jax.experimental.pallas import tpu_sc as plsc

N = 10000
E = 160000
D = 256
LAT = 256
NEG = 0.2

NC = 2
NS = 16
L = 16
NW = NC * NS

NPAD = 10240
RNG = NPAD // NW
EPAD = 163840
EW1 = EPAD // NW
G = 128
C = 1024
CAP = C + 2 * G
NCH = EPAD // C

BN = 1024

_SC_PARAMS = pltpu.CompilerParams(needs_layout_passes=False)


def _elu(x):
    return jnp.where(x > 0, x, jnp.exp(jnp.minimum(x, 0.0)) - 1.0)


def _alpha_block(h, a_src_ref, a_dst_ref, i, nblk, asrc_ref, adst_ref,
                 shift_ref, mx_ref):
    av = jnp.dot(h, a_src_ref[...], preferred_element_type=jnp.float32)
    bv = jnp.dot(h, a_dst_ref[...], preferred_element_type=jnp.float32)
    asrc_ref[...] = av
    adst_ref[...] = bv
    am = jnp.max(av)
    bm = jnp.max(bv)

    @pl.when(i == 0)
    def _():
        mx_ref[0] = am
        mx_ref[1] = bm

    @pl.when(i > 0)
    def _():
        mx_ref[0] = jnp.maximum(mx_ref[0], am)
        mx_ref[1] = jnp.maximum(mx_ref[1], bm)

    @pl.when(i == nblk - 1)
    def _():
        shift_ref[...] = jnp.full(
            (1, 1),
            jnp.maximum(mx_ref[0], 0.0) + jnp.maximum(mx_ref[1], 0.0),
            jnp.float32)


def _tc1_body(x_ref, w_ref, a_src_ref, a_dst_ref,
              h_ref, asrc_ref, adst_ref, shift_ref, mx_ref):
    h = jnp.dot(x_ref[...], w_ref[...], preferred_element_type=jnp.float32)
    h_ref[...] = h
    _alpha_block(h, a_src_ref, a_dst_ref, pl.program_id(0), pl.num_programs(0),
                 asrc_ref, adst_ref, shift_ref, mx_ref)


def _tc1(x, W, a_src, a_dst):
    nblk = NPAD // BN
    return pl.pallas_call(
        _tc1_body,
        grid=(nblk,),
        in_specs=[
            pl.BlockSpec((BN, D), lambda i: (i, 0)),
            pl.BlockSpec((D, D), lambda i: (0, 0)),
            pl.BlockSpec((D, 1), lambda i: (0, 0)),
            pl.BlockSpec((D, 1), lambda i: (0, 0)),
        ],
        out_specs=[
            pl.BlockSpec((BN, D), lambda i: (i, 0)),
            pl.BlockSpec((BN, 1), lambda i: (i, 0)),
            pl.BlockSpec((BN, 1), lambda i: (i, 0)),
            pl.BlockSpec((1, 1), lambda i: (0, 0)),
        ],
        out_shape=[
            jax.ShapeDtypeStruct((NPAD, D), jnp.float32),
            jax.ShapeDtypeStruct((NPAD, 1), jnp.float32),
            jax.ShapeDtypeStruct((NPAD, 1), jnp.float32),
            jax.ShapeDtypeStruct((1, 1), jnp.float32),
        ],
        scratch_shapes=[pltpu.SMEM((2,), jnp.float32)],
    )(x, W, a_src, a_dst)


def _tc2_body(agg_ref, den_ref, b_ref, g_ref, lb_ref, w_ref,
              a_src_ref, a_dst_ref,
              h_ref, asrc_ref, adst_ref, shift_ref, mx_ref):
    o = agg_ref[...] / (den_ref[...] + 1e-16) + b_ref[...]
    o = _elu(o)
    mu = jnp.mean(o, axis=-1, keepdims=True)
    var = jnp.mean((o - mu) ** 2, axis=-1, keepdims=True)
    hn = (o - mu) / jnp.sqrt(var + 1e-5) * g_ref[...] + lb_ref[...]
    h = jnp.dot(hn, w_ref[...], preferred_element_type=jnp.float32)
    h_ref[...] = h
    _alpha_block(h, a_src_ref, a_dst_ref, pl.program_id(0), pl.num_programs(0),
                 asrc_ref, adst_ref, shift_ref, mx_ref)


def _tc2(agg, den, b, g, lb, W, a_src, a_dst):
    nblk = NPAD // BN
    return pl.pallas_call(
        _tc2_body,
        grid=(nblk,),
        in_specs=[
            pl.BlockSpec((BN, D), lambda i: (i, 0)),
            pl.BlockSpec((BN, 1), lambda i: (i, 0)),
            pl.BlockSpec((1, D), lambda i: (0, 0)),
            pl.BlockSpec((1, D), lambda i: (0, 0)),
            pl.BlockSpec((1, D), lambda i: (0, 0)),
            pl.BlockSpec((D, D), lambda i: (0, 0)),
            pl.BlockSpec((D, 1), lambda i: (0, 0)),
            pl.BlockSpec((D, 1), lambda i: (0, 0)),
        ],
        out_specs=[
            pl.BlockSpec((BN, D), lambda i: (i, 0)),
            pl.BlockSpec((BN, 1), lambda i: (i, 0)),
            pl.BlockSpec((BN, 1), lambda i: (i, 0)),
            pl.BlockSpec((1, 1), lambda i: (0, 0)),
        ],
        out_shape=[
            jax.ShapeDtypeStruct((NPAD, D), jnp.float32),
            jax.ShapeDtypeStruct((NPAD, 1), jnp.float32),
            jax.ShapeDtypeStruct((NPAD, 1), jnp.float32),
            jax.ShapeDtypeStruct((1, 1), jnp.float32),
        ],
        scratch_shapes=[pltpu.SMEM((2,), jnp.float32)],
    )(agg, den, b, g, lb, W, a_src, a_dst)


def _tc3_body(agg_ref, den_ref, b_ref, ew1_ref, eb1_ref, ew2_ref, eb2_ref,
              pw_ref, pb_ref, x_ref, tw_ref, tb_ref, zp_ref, zt_ref):
    o = agg_ref[...] / (den_ref[...] + 1e-16) + b_ref[...]
    o = _elu(o)
    he = jnp.maximum(
        jnp.dot(o, ew1_ref[...], preferred_element_type=jnp.float32)
        + eb1_ref[...], 0.0)
    he = jnp.dot(he, ew2_ref[...], preferred_element_type=jnp.float32) \
        + eb2_ref[...]
    zp_ref[...] = jnp.dot(he, pw_ref[...],
                          preferred_element_type=jnp.float32) + pb_ref[...]
    zt_ref[...] = jnp.dot(x_ref[...], tw_ref[...],
                          preferred_element_type=jnp.float32) + tb_ref[...]


def _tc3(agg, den, b, ew1, eb1, ew2, eb2, pw, pb, x, tw, tb):
    nblk = NPAD // BN
    full = lambda r, c: pl.BlockSpec((r, c), lambda i: (0, 0))
    blk = lambda c: pl.BlockSpec((BN, c), lambda i: (i, 0))
    return pl.pallas_call(
        _tc3_body,
        grid=(nblk,),
        in_specs=[
            blk(D), pl.BlockSpec((BN, 1), lambda i: (i, 0)), full(1, D),
            full(D, D), full(1, D), full(D, D), full(1, D),
            full(D, LAT), full(1, LAT),
            blk(D), full(D, LAT), full(1, LAT),
        ],
        out_specs=[blk(LAT), blk(LAT)],
        out_shape=[
            jax.ShapeDtypeStruct((NPAD, LAT), jnp.float32),
            jax.ShapeDtypeStruct((NPAD, LAT), jnp.float32),
        ],
    )(agg, den, b, ew1, eb1, ew2, eb2, pw, pb, x, tw, tb)


def _sc_ex_body(src_hbm, dst_hbm, asrc_hbm, adst_hbm, shift_hbm,
                ex_hbm, src_v, dst_v, av, bv, shift_v, sem):
    c = lax.axis_index("c")
    s = lax.axis_index("s")
    w = s * NC + c
    base = w * EW1

    pltpu.sync_copy(shift_hbm, shift_v)
    pltpu.sync_copy(src_hbm.at[pl.ds(base, EW1)], src_v)
    pltpu.sync_copy(dst_hbm.at[pl.ds(base, EW1)], dst_v)

    def _gather(i, carry):
        d1 = pltpu.async_copy(asrc_hbm.at[src_v.at[pl.ds(i * G, G)]],
                              av.at[pl.ds(i * G, G)], sem)
        d2 = pltpu.async_copy(adst_hbm.at[dst_v.at[pl.ds(i * G, G)]],
                              bv.at[pl.ds(i * G, G)], sem)
        d1.wait()
        d2.wait()
        return carry

    lax.fori_loop(0, EW1 // G, _gather, 0)

    shift = shift_v[...]

    def _ex(i, carry):
        a = av[pl.ds(i * L, L)] + bv[pl.ds(i * L, L)]
        e = jnp.where(a >= 0, a, NEG * a)
        av[pl.ds(i * L, L)] = jnp.exp(e - shift)
        return carry

    lax.fori_loop(0, EW1 // L, _ex, 0)
    pltpu.sync_copy(av, ex_hbm.at[pl.ds(base, EW1)])


@functools.cache
def _sc_ex_kernel():
    return pl.kernel(
        _sc_ex_body,
        compiler_params=_SC_PARAMS,
        out_type=[jax.ShapeDtypeStruct((EPAD,), jnp.float32)],
        mesh=plsc.VectorSubcoreMesh(core_axis_name="c", subcore_axis_name="s",
                                    num_cores=NC, num_subcores=NS),
        scratch_types=[
            pltpu.VMEM((EW1,), jnp.int32),
            pltpu.VMEM((EW1,), jnp.int32),
            pltpu.VMEM((EW1,), jnp.float32),
            pltpu.VMEM((EW1,), jnp.float32),
            pltpu.VMEM((L,), jnp.float32),
            pltpu.SemaphoreType.DMA,
        ],
    )


def _sc_agg_body(src_hbm, dst_hbm, ex_hbm, h_hbm,
                 agg_hbm, den_hbm,
                 srcc, dstc, exc, crow, csrc, cex, gsrc, grow, gex,
                 rows, acc, denloc, sem0, sem1, gsem):
    c = lax.axis_index("c")
    s = lax.axis_index("s")
    w = s * NC + c
    lo = w * RNG

    zf = jnp.zeros((L,), jnp.float32)
    zi = jnp.zeros((L,), jnp.int32)
    dummy = jnp.full((L,), RNG, jnp.int32)

    def _zacc(i, carry):
        r = i // (D // L)
        col = (i % (D // L)) * L
        acc[r, pl.ds(col, L)] = zf
        return carry

    lax.fori_loop(0, (RNG + 8) * (D // L), _zacc, 0)

    def _zden(i, carry):
        denloc[pl.ds(i * L, L)] = zf
        return carry

    lax.fori_loop(0, (RNG + 16) // L, _zden, 0)

    def _accum_from(rowsrc_ref, exsrc_ref, off):
        def _acc16(r16, carry2):
            cb = off + r16 * L
            rb = r16 * L
            exw = exsrc_ref[pl.ds(cb, L)]
            rv = rowsrc_ref[pl.ds(cb, L)]
            plsc.addupdate_scatter(denloc, [rv], exw)
            for k in range(L):
                wv = jnp.full((L,), exw[k], jnp.float32)
                row = rv[k]
                for j in range(D // L):
                    plsc.addupdate(
                        acc.at[row, pl.ds(j * L, L)],
                        rows[rb + k, pl.ds(j * L, L)] * wv)
            return carry2

        lax.fori_loop(0, G // L, _acc16, 0)

    def _retire(pend):
        @pl.when(pend == 1)
        def _():
            pltpu.make_async_copy(h_hbm.at[gsrc], rows, gsem).wait()
            _accum_from(grow, gex, 0)

    def _start_chunk(ch, b, sem):
        base = ch * C
        pltpu.async_copy(src_hbm.at[pl.ds(base, C)], srcc.at[b], sem)
        pltpu.async_copy(dst_hbm.at[pl.ds(base, C)], dstc.at[b], sem)
        pltpu.async_copy(ex_hbm.at[pl.ds(base, C)], exc.at[b], sem)

    def _wait_chunk(ch, b, sem):
        base = ch * C
        pltpu.make_async_copy(src_hbm.at[pl.ds(base, C)], srcc.at[b],
                              sem).wait()
        pltpu.make_async_copy(dst_hbm.at[pl.ds(base, C)], dstc.at[b],
                              sem).wait()
        pltpu.make_async_copy(ex_hbm.at[pl.ds(base, C)], exc.at[b],
                              sem).wait()

    def _scan_chunk(b, cnt):
        def _scan(i, cn):
            for u in range(4):
                ii = i * 4 + u
                d = dstc[b, pl.ds(ii * L, L)]
                m = (d >= lo) & (d < lo + RNG)
                plsc.store_compressed(crow.at[pl.ds(cn, L)], d - lo, mask=m)
                plsc.store_compressed(csrc.at[pl.ds(cn, L)],
                                      srcc[b, pl.ds(ii * L, L)], mask=m)
                plsc.store_compressed(cex.at[pl.ds(cn, L)],
                                      exc[b, pl.ds(ii * L, L)], mask=m)
                cn = cn + plsc.all_reduce_population_count(m)[0]
            return cn

        return lax.fori_loop(0, C // L // 4, _scan, cnt)

    def _boundary(cnt, pend):
        _retire(pend)
        nd = cnt // G

        def _extra(g, carry):
            pltpu.async_copy(h_hbm.at[csrc.at[pl.ds(g * G, G)]], rows,
                             gsem).wait()
            _accum_from(crow, cex, g * G)
            return carry

        lax.fori_loop(1, jnp.maximum(nd, 1), _extra, 0)

        @pl.when(nd >= 1)
        def _():
            for t in range(G // L):
                gsrc[pl.ds(t * L, L)] = csrc[pl.ds(t * L, L)]
                grow[pl.ds(t * L, L)] = crow[pl.ds(t * L, L)]
                gex[pl.ds(t * L, L)] = cex[pl.ds(t * L, L)]
            pltpu.async_copy(h_hbm.at[gsrc], rows, gsem)

        for t in range(G // L):
            crow[pl.ds(t * L, L)] = crow[pl.ds(nd * G + t * L, L)]
            csrc[pl.ds(t * L, L)] = csrc[pl.ds(nd * G + t * L, L)]
            cex[pl.ds(t * L, L)] = cex[pl.ds(nd * G + t * L, L)]
        return cnt - nd * G, (nd >= 1).astype(jnp.int32)

    _start_chunk(0, 0, sem0)

    def _pair(p, state):
        cnt, pend = state
        ch0 = 2 * p
        _start_chunk(ch0 + 1, 1, sem1)
        _wait_chunk(ch0, 0, sem0)
        cnt = _scan_chunk(0, cnt)
        cnt, pend = _boundary(cnt, pend)

        @pl.when(ch0 + 2 < NCH)
        def _():
            _start_chunk(ch0 + 2, 0, sem0)

        _wait_chunk(ch0 + 1, 1, sem1)
        cnt = _scan_chunk(1, cnt)
        return _boundary(cnt, pend)

    cnt, pend = lax.fori_loop(0, NCH // 2, _pair,
                              (jnp.int32(0), jnp.int32(0)))
    _retire(pend)

    for t in range(G // L):
        crow[pl.ds(cnt + t * L, L)] = dummy
        csrc[pl.ds(cnt + t * L, L)] = zi
        cex[pl.ds(cnt + t * L, L)] = zf

    @pl.when(cnt > 0)
    def _():
        pltpu.async_copy(h_hbm.at[csrc.at[pl.ds(0, G)]], rows, gsem).wait()
        _accum_from(crow, cex, 0)

    pltpu.sync_copy(acc.at[pl.ds(0, RNG)], agg_hbm.at[pl.ds(lo, RNG)])
    pltpu.sync_copy(denloc.at[pl.ds(0, RNG)], den_hbm.at[pl.ds(lo, RNG)])


@functools.cache
def _sc_agg_kernel():
    return pl.kernel(
        _sc_agg_body,
        compiler_params=_SC_PARAMS,
        out_type=[
            jax.ShapeDtypeStruct((NPAD, D), jnp.float32),
            jax.ShapeDtypeStruct((NPAD,), jnp.float32),
        ],
        mesh=plsc.VectorSubcoreMesh(core_axis_name="c", subcore_axis_name="s",
                                    num_cores=NC, num_subcores=NS),
        scratch_types=[
            pltpu.VMEM((2, C), jnp.int32),
            pltpu.VMEM((2, C), jnp.int32),
            pltpu.VMEM((2, C), jnp.float32),
            pltpu.VMEM((CAP,), jnp.int32),
            pltpu.VMEM((CAP,), jnp.int32),
            pltpu.VMEM((CAP,), jnp.float32),
            pltpu.VMEM((G,), jnp.int32),
            pltpu.VMEM((G,), jnp.int32),
            pltpu.VMEM((G,), jnp.float32),
            pltpu.VMEM((G, D), jnp.float32),
            pltpu.VMEM((RNG + 8, D), jnp.float32),
            pltpu.VMEM((RNG + 16,), jnp.float32),
            pltpu.SemaphoreType.DMA,
            pltpu.SemaphoreType.DMA,
            pltpu.SemaphoreType.DMA,
        ],
    )


def kernel(x, edge_index, W0, a_src0, a_dst0, b0, ln0_g, ln0_b, W1, a_src1,
           a_dst1, b1, encW1, encb1, encW2, encb2, predW, predb, tgtW, tgtb):
    src = edge_index[0].astype(jnp.int32)
    dst = edge_index[1].astype(jnp.int32)
    src_p = jnp.concatenate([src, jnp.zeros((EPAD - E,), jnp.int32)])
    dst_p = jnp.concatenate([dst, jnp.full((EPAD - E,), N, jnp.int32)])
    x_pad = jnp.zeros((NPAD, D), jnp.float32).at[:N].set(x)

    h0, asrc0, adst0, shift0 = _tc1(x_pad, W0, a_src0.reshape(D, 1),
                                    a_dst0.reshape(D, 1))
    sv0 = jnp.broadcast_to(shift0.reshape(()), (L,))
    ex0, = _sc_ex_kernel()(src_p, dst_p, asrc0.reshape(-1), adst0.reshape(-1),
                           sv0)
    agg0, den0 = _sc_agg_kernel()(src_p, dst_p, ex0, h0)

    h1, asrc1, adst1, shift1 = _tc2(agg0, den0.reshape(NPAD, 1),
                                    b0.reshape(1, D), ln0_g.reshape(1, D),
                                    ln0_b.reshape(1, D), W1,
                                    a_src1.reshape(D, 1), a_dst1.reshape(D, 1))
    sv1 = jnp.broadcast_to(shift1.reshape(()), (L,))
    ex1, = _sc_ex_kernel()(src_p, dst_p, asrc1.reshape(-1), adst1.reshape(-1),
                           sv1)
    agg1, den1 = _sc_agg_kernel()(src_p, dst_p, ex1, h1)

    zp, zt = _tc3(agg1, den1.reshape(NPAD, 1), b1.reshape(1, D),
                  encW1, encb1.reshape(1, D), encW2, encb2.reshape(1, D),
                  predW, predb.reshape(1, LAT), x_pad, tgtW,
                  tgtb.reshape(1, LAT))
    return (zp[:N], zt[:N])


# speedup vs baseline: 1.0145x; 1.0145x over previous
import functools

import jax
import jax.numpy as jnp
from jax import lax
from jax.experimental import pallas as pl
from jax.experimental.pallas import tpu as pltpu
from jax.experimental.pallas import tpu_sc as plsc

N = 10000
E = 160000
D = 256
LAT = 256
NEG = 0.2

NC = 2
NS = 16
L = 16
NW = NC * NS

NPAD = 10240
RNG = NPAD // NW
EPAD = 163840
EW1 = EPAD // NW
G = 128
C = 1024
CAP = C + 2 * G
NCH = EPAD // C

BN = 1024

_SC_PARAMS = pltpu.CompilerParams(needs_layout_passes=False)


def _elu(x):
    return jnp.where(x > 0, x, jnp.exp(jnp.minimum(x, 0.0)) - 1.0)


def _alpha_block(h, a_src_ref, a_dst_ref, i, nblk, asrc_ref, adst_ref,
                 shift_ref, mx_ref):
    av = jnp.dot(h, a_src_ref[...], preferred_element_type=jnp.float32)
    bv = jnp.dot(h, a_dst_ref[...], preferred_element_type=jnp.float32)
    asrc_ref[...] = av
    adst_ref[...] = bv
    am = jnp.max(av)
    bm = jnp.max(bv)

    @pl.when(i == 0)
    def _():
        mx_ref[0] = am
        mx_ref[1] = bm

    @pl.when(i > 0)
    def _():
        mx_ref[0] = jnp.maximum(mx_ref[0], am)
        mx_ref[1] = jnp.maximum(mx_ref[1], bm)

    @pl.when(i == nblk - 1)
    def _():
        shift_ref[...] = jnp.full(
            (1, 1),
            jnp.maximum(mx_ref[0], 0.0) + jnp.maximum(mx_ref[1], 0.0),
            jnp.float32)


def _tc1_body(x_ref, w_ref, a_src_ref, a_dst_ref,
              h_ref, asrc_ref, adst_ref, shift_ref, mx_ref):
    h = jnp.dot(x_ref[...], w_ref[...], preferred_element_type=jnp.float32)
    h_ref[...] = h
    _alpha_block(h, a_src_ref, a_dst_ref, pl.program_id(0), pl.num_programs(0),
                 asrc_ref, adst_ref, shift_ref, mx_ref)


def _tc1(x, W, a_src, a_dst):
    nblk = NPAD // BN
    return pl.pallas_call(
        _tc1_body,
        grid=(nblk,),
        in_specs=[
            pl.BlockSpec((BN, D), lambda i: (i, 0)),
            pl.BlockSpec((D, D), lambda i: (0, 0)),
            pl.BlockSpec((D, 1), lambda i: (0, 0)),
            pl.BlockSpec((D, 1), lambda i: (0, 0)),
        ],
        out_specs=[
            pl.BlockSpec((BN, D), lambda i: (i, 0)),
            pl.BlockSpec((BN, 1), lambda i: (i, 0)),
            pl.BlockSpec((BN, 1), lambda i: (i, 0)),
            pl.BlockSpec((1, 1), lambda i: (0, 0)),
        ],
        out_shape=[
            jax.ShapeDtypeStruct((NPAD, D), jnp.float32),
            jax.ShapeDtypeStruct((NPAD, 1), jnp.float32),
            jax.ShapeDtypeStruct((NPAD, 1), jnp.float32),
            jax.ShapeDtypeStruct((1, 1), jnp.float32),
        ],
        scratch_shapes=[pltpu.SMEM((2,), jnp.float32)],
    )(x, W, a_src, a_dst)


def _tc2_body(agg_ref, den_ref, b_ref, g_ref, lb_ref, w_ref,
              a_src_ref, a_dst_ref,
              h_ref, asrc_ref, adst_ref, shift_ref, mx_ref):
    o = agg_ref[...] / (den_ref[...] + 1e-16) + b_ref[...]
    o = _elu(o)
    mu = jnp.mean(o, axis=-1, keepdims=True)
    var = jnp.mean((o - mu) ** 2, axis=-1, keepdims=True)
    hn = (o - mu) / jnp.sqrt(var + 1e-5) * g_ref[...] + lb_ref[...]
    h = jnp.dot(hn, w_ref[...], preferred_element_type=jnp.float32)
    h_ref[...] = h
    _alpha_block(h, a_src_ref, a_dst_ref, pl.program_id(0), pl.num_programs(0),
                 asrc_ref, adst_ref, shift_ref, mx_ref)


def _tc2(agg, den, b, g, lb, W, a_src, a_dst):
    nblk = NPAD // BN
    return pl.pallas_call(
        _tc2_body,
        grid=(nblk,),
        in_specs=[
            pl.BlockSpec((BN, D), lambda i: (i, 0)),
            pl.BlockSpec((BN, 1), lambda i: (i, 0)),
            pl.BlockSpec((1, D), lambda i: (0, 0)),
            pl.BlockSpec((1, D), lambda i: (0, 0)),
            pl.BlockSpec((1, D), lambda i: (0, 0)),
            pl.BlockSpec((D, D), lambda i: (0, 0)),
            pl.BlockSpec((D, 1), lambda i: (0, 0)),
            pl.BlockSpec((D, 1), lambda i: (0, 0)),
        ],
        out_specs=[
            pl.BlockSpec((BN, D), lambda i: (i, 0)),
            pl.BlockSpec((BN, 1), lambda i: (i, 0)),
            pl.BlockSpec((BN, 1), lambda i: (i, 0)),
            pl.BlockSpec((1, 1), lambda i: (0, 0)),
        ],
        out_shape=[
            jax.ShapeDtypeStruct((NPAD, D), jnp.float32),
            jax.ShapeDtypeStruct((NPAD, 1), jnp.float32),
            jax.ShapeDtypeStruct((NPAD, 1), jnp.float32),
            jax.ShapeDtypeStruct((1, 1), jnp.float32),
        ],
        scratch_shapes=[pltpu.SMEM((2,), jnp.float32)],
    )(agg, den, b, g, lb, W, a_src, a_dst)


def _tc3_body(agg_ref, den_ref, b_ref, ew1_ref, eb1_ref, ew2_ref, eb2_ref,
              pw_ref, pb_ref, x_ref, tw_ref, tb_ref, zp_ref, zt_ref):
    o = agg_ref[...] / (den_ref[...] + 1e-16) + b_ref[...]
    o = _elu(o)
    he = jnp.maximum(
        jnp.dot(o, ew1_ref[...], preferred_element_type=jnp.float32)
        + eb1_ref[...], 0.0)
    he = jnp.dot(he, ew2_ref[...], preferred_element_type=jnp.float32) \
        + eb2_ref[...]
    zp_ref[...] = jnp.dot(he, pw_ref[...],
                          preferred_element_type=jnp.float32) + pb_ref[...]
    zt_ref[...] = jnp.dot(x_ref[...], tw_ref[...],
                          preferred_element_type=jnp.float32) + tb_ref[...]


def _tc3(agg, den, b, ew1, eb1, ew2, eb2, pw, pb, x, tw, tb):
    nblk = NPAD // BN
    full = lambda r, c: pl.BlockSpec((r, c), lambda i: (0, 0))
    blk = lambda c: pl.BlockSpec((BN, c), lambda i: (i, 0))
    return pl.pallas_call(
        _tc3_body,
        grid=(nblk,),
        in_specs=[
            blk(D), pl.BlockSpec((BN, 1), lambda i: (i, 0)), full(1, D),
            full(D, D), full(1, D), full(D, D), full(1, D),
            full(D, LAT), full(1, LAT),
            blk(D), full(D, LAT), full(1, LAT),
        ],
        out_specs=[blk(LAT), blk(LAT)],
        out_shape=[
            jax.ShapeDtypeStruct((NPAD, LAT), jnp.float32),
            jax.ShapeDtypeStruct((NPAD, LAT), jnp.float32),
        ],
    )(agg, den, b, ew1, eb1, ew2, eb2, pw, pb, x, tw, tb)


def _sc_ex_body(src_hbm, dst_hbm, asrc_hbm, adst_hbm, shift_hbm,
                ex_hbm, src_v, dst_v, av, bv, shift_v, sem):
    c = lax.axis_index("c")
    s = lax.axis_index("s")
    w = s * NC + c
    base = w * EW1

    pltpu.sync_copy(shift_hbm, shift_v)
    pltpu.sync_copy(src_hbm.at[pl.ds(base, EW1)], src_v)
    pltpu.sync_copy(dst_hbm.at[pl.ds(base, EW1)], dst_v)

    def _gather(i, carry):
        d1 = pltpu.async_copy(asrc_hbm.at[src_v.at[pl.ds(i * G, G)]],
                              av.at[pl.ds(i * G, G)], sem)
        d2 = pltpu.async_copy(adst_hbm.at[dst_v.at[pl.ds(i * G, G)]],
                              bv.at[pl.ds(i * G, G)], sem)
        d1.wait()
        d2.wait()
        return carry

    lax.fori_loop(0, EW1 // G, _gather, 0)

    shift = shift_v[...]

    def _ex(i, carry):
        a = av[pl.ds(i * L, L)] + bv[pl.ds(i * L, L)]
        e = jnp.where(a >= 0, a, NEG * a)
        av[pl.ds(i * L, L)] = jnp.exp(e - shift)
        return carry

    lax.fori_loop(0, EW1 // L, _ex, 0)
    pltpu.sync_copy(av, ex_hbm.at[pl.ds(base, EW1)])


@functools.cache
def _sc_ex_kernel():
    return pl.kernel(
        _sc_ex_body,
        compiler_params=_SC_PARAMS,
        out_type=[jax.ShapeDtypeStruct((EPAD,), jnp.float32)],
        mesh=plsc.VectorSubcoreMesh(core_axis_name="c", subcore_axis_name="s",
                                    num_cores=NC, num_subcores=NS),
        scratch_types=[
            pltpu.VMEM((EW1,), jnp.int32),
            pltpu.VMEM((EW1,), jnp.int32),
            pltpu.VMEM((EW1,), jnp.float32),
            pltpu.VMEM((EW1,), jnp.float32),
            pltpu.VMEM((L,), jnp.float32),
            pltpu.SemaphoreType.DMA,
        ],
    )


def _sc_agg_body(src_hbm, dst_hbm, ex_hbm, h_hbm,
                 agg_hbm, den_hbm,
                 srcc, dstc, exc, crow, csrc, cex, gsrc, grow, gex,
                 rows, acc, denloc, sem0, sem1, gsem):
    c = lax.axis_index("c")
    s = lax.axis_index("s")
    w = s * NC + c
    lo = w * RNG

    zf = jnp.zeros((L,), jnp.float32)
    zi = jnp.zeros((L,), jnp.int32)
    dummy = jnp.full((L,), RNG, jnp.int32)

    def _zacc(i, carry):
        r = i // (D // L)
        col = (i % (D // L)) * L
        acc[r, pl.ds(col, L)] = zf
        return carry

    lax.fori_loop(0, (RNG + 8) * (D // L), _zacc, 0)

    def _zden(i, carry):
        denloc[pl.ds(i * L, L)] = zf
        return carry

    lax.fori_loop(0, (RNG + 16) // L, _zden, 0)

    def _accum_from(rowsrc_ref, exsrc_ref, off):
        def _acc16(r16, carry2):
            cb = off + r16 * L
            rb = r16 * L
            exw = exsrc_ref[pl.ds(cb, L)]
            rv = rowsrc_ref[pl.ds(cb, L)]
            plsc.addupdate_scatter(denloc, [rv], exw)
            for k in range(L):
                wv = jnp.full((L,), exw[k], jnp.float32)
                row = rv[k]
                for j in range(D // L):
                    plsc.addupdate(
                        acc.at[row, pl.ds(j * L, L)],
                        rows[rb + k, pl.ds(j * L, L)] * wv)
            return carry2

        lax.fori_loop(0, G // L, _acc16, 0)

    def _retire(pend):
        @pl.when(pend == 1)
        def _():
            pltpu.make_async_copy(h_hbm.at[gsrc], rows, gsem).wait()
            _accum_from(grow, gex, 0)

    def _start_chunk(ch, b, sem):
        base = ch * C
        pltpu.async_copy(src_hbm.at[pl.ds(base, C)], srcc.at[b], sem)
        pltpu.async_copy(dst_hbm.at[pl.ds(base, C)], dstc.at[b], sem)
        pltpu.async_copy(ex_hbm.at[pl.ds(base, C)], exc.at[b], sem)

    def _wait_chunk(ch, b, sem):
        base = ch * C
        pltpu.make_async_copy(src_hbm.at[pl.ds(base, C)], srcc.at[b],
                              sem).wait()
        pltpu.make_async_copy(dst_hbm.at[pl.ds(base, C)], dstc.at[b],
                              sem).wait()
        pltpu.make_async_copy(ex_hbm.at[pl.ds(base, C)], exc.at[b],
                              sem).wait()

    def _scan_chunk(b, cnt):
        def _scan(i, cn):
            for u in range(4):
                ii = i * 4 + u
                d = dstc[b, pl.ds(ii * L, L)]
                m = (d >= lo) & (d < lo + RNG)
                plsc.store_compressed(crow.at[pl.ds(cn, L)], d - lo, mask=m)
                plsc.store_compressed(csrc.at[pl.ds(cn, L)],
                                      srcc[b, pl.ds(ii * L, L)], mask=m)
                plsc.store_compressed(cex.at[pl.ds(cn, L)],
                                      exc[b, pl.ds(ii * L, L)], mask=m)
                cn = cn + plsc.all_reduce_population_count(m)[0]
            return cn

        return lax.fori_loop(0, C // L // 4, _scan, cnt)

    def _boundary(cnt, pend):
        nd = cnt // G

        @pl.when(nd >= 1)
        def _():
            _retire(pend)

            def _extra(g, carry):
                pltpu.async_copy(h_hbm.at[csrc.at[pl.ds(g * G, G)]], rows,
                                 gsem).wait()
                _accum_from(crow, cex, g * G)
                return carry

            lax.fori_loop(1, jnp.maximum(nd, 1), _extra, 0)

            for t in range(G // L):
                gsrc[pl.ds(t * L, L)] = csrc[pl.ds(t * L, L)]
                grow[pl.ds(t * L, L)] = crow[pl.ds(t * L, L)]
                gex[pl.ds(t * L, L)] = cex[pl.ds(t * L, L)]
            pltpu.async_copy(h_hbm.at[gsrc], rows, gsem)

            for t in range(G // L):
                crow[pl.ds(t * L, L)] = crow[pl.ds(nd * G + t * L, L)]
                csrc[pl.ds(t * L, L)] = csrc[pl.ds(nd * G + t * L, L)]
                cex[pl.ds(t * L, L)] = cex[pl.ds(nd * G + t * L, L)]

        pend_new = jnp.where(nd >= 1, jnp.int32(1), pend)
        return cnt - nd * G, pend_new

    _start_chunk(0, 0, sem0)

    def _pair(p, state):
        cnt, pend = state
        ch0 = 2 * p
        _start_chunk(ch0 + 1, 1, sem1)
        _wait_chunk(ch0, 0, sem0)
        cnt = _scan_chunk(0, cnt)
        cnt, pend = _boundary(cnt, pend)

        @pl.when(ch0 + 2 < NCH)
        def _():
            _start_chunk(ch0 + 2, 0, sem0)

        _wait_chunk(ch0 + 1, 1, sem1)
        cnt = _scan_chunk(1, cnt)
        return _boundary(cnt, pend)

    cnt, pend = lax.fori_loop(0, NCH // 2, _pair,
                              (jnp.int32(0), jnp.int32(0)))
    _retire(pend)

    for t in range(G // L):
        crow[pl.ds(cnt + t * L, L)] = dummy
        csrc[pl.ds(cnt + t * L, L)] = zi
        cex[pl.ds(cnt + t * L, L)] = zf

    @pl.when(cnt > 0)
    def _():
        pltpu.async_copy(h_hbm.at[csrc.at[pl.ds(0, G)]], rows, gsem).wait()
        _accum_from(crow, cex, 0)

    pltpu.sync_copy(acc.at[pl.ds(0, RNG)], agg_hbm.at[pl.ds(lo, RNG)])
    pltpu.sync_copy(denloc.at[pl.ds(0, RNG)], den_hbm.at[pl.ds(lo, RNG)])


@functools.cache
def _sc_agg_kernel():
    return pl.kernel(
        _sc_agg_body,
        compiler_params=_SC_PARAMS,
        out_type=[
            jax.ShapeDtypeStruct((NPAD, D), jnp.float32),
            jax.ShapeDtypeStruct((NPAD,), jnp.float32),
        ],
        mesh=plsc.VectorSubcoreMesh(core_axis_name="c", subcore_axis_name="s",
                                    num_cores=NC, num_subcores=NS),
        scratch_types=[
            pltpu.VMEM((2, C), jnp.int32),
            pltpu.VMEM((2, C), jnp.int32),
            pltpu.VMEM((2, C), jnp.float32),
            pltpu.VMEM((CAP,), jnp.int32),
            pltpu.VMEM((CAP,), jnp.int32),
            pltpu.VMEM((CAP,), jnp.float32),
            pltpu.VMEM((G,), jnp.int32),
            pltpu.VMEM((G,), jnp.int32),
            pltpu.VMEM((G,), jnp.float32),
            pltpu.VMEM((G, D), jnp.float32),
            pltpu.VMEM((RNG + 8, D), jnp.float32),
            pltpu.VMEM((RNG + 16,), jnp.float32),
            pltpu.SemaphoreType.DMA,
            pltpu.SemaphoreType.DMA,
            pltpu.SemaphoreType.DMA,
        ],
    )


def kernel(x, edge_index, W0, a_src0, a_dst0, b0, ln0_g, ln0_b, W1, a_src1,
           a_dst1, b1, encW1, encb1, encW2, encb2, predW, predb, tgtW, tgtb):
    src = edge_index[0].astype(jnp.int32)
    dst = edge_index[1].astype(jnp.int32)
    src_p = jnp.concatenate([src, jnp.zeros((EPAD - E,), jnp.int32)])
    dst_p = jnp.concatenate([dst, jnp.full((EPAD - E,), N, jnp.int32)])
    x_pad = jnp.zeros((NPAD, D), jnp.float32).at[:N].set(x)

    h0, asrc0, adst0, shift0 = _tc1(x_pad, W0, a_src0.reshape(D, 1),
                                    a_dst0.reshape(D, 1))
    sv0 = jnp.broadcast_to(shift0.reshape(()), (L,))
    ex0, = _sc_ex_kernel()(src_p, dst_p, asrc0.reshape(-1), adst0.reshape(-1),
                           sv0)
    agg0, den0 = _sc_agg_kernel()(src_p, dst_p, ex0, h0)

    h1, asrc1, adst1, shift1 = _tc2(agg0, den0.reshape(NPAD, 1),
                                    b0.reshape(1, D), ln0_g.reshape(1, D),
                                    ln0_b.reshape(1, D), W1,
                                    a_src1.reshape(D, 1), a_dst1.reshape(D, 1))
    sv1 = jnp.broadcast_to(shift1.reshape(()), (L,))
    ex1, = _sc_ex_kernel()(src_p, dst_p, asrc1.reshape(-1), adst1.reshape(-1),
                           sv1)
    agg1, den1 = _sc_agg_kernel()(src_p, dst_p, ex1, h1)

    zp, zt = _tc3(agg1, den1.reshape(NPAD, 1), b1.reshape(1, D),
                  encW1, encb1.reshape(1, D), encW2, encb2.reshape(1, D),
                  predW, predb.reshape(1, LAT), x_pad, tgtW,
                  tgtb.reshape(1, LAT))
    return (zp[:N], zt[:N])


# speedup vs baseline: 1.3953x; 1.3753x over previous
import functools

import jax
import jax.numpy as jnp
from jax import lax
from jax.experimental import pallas as pl
from jax.experimental.pallas import tpu as pltpu
from jax.experimental.pallas import tpu_sc as plsc

N = 10000
E = 160000
D = 256
LAT = 256
NEG = 0.2

NC = 2
NS = 16
L = 16
NW = NC * NS

NPAD = 10240
RNG = NPAD // NW
EPAD = 163840
EW1 = EPAD // NW
G = 128
C = 1024
CAP = C + 2 * G
NCH = EPAD // C

BN = 1024

_SC_PARAMS = pltpu.CompilerParams(needs_layout_passes=False)


def _elu(x):
    return jnp.where(x > 0, x, jnp.exp(jnp.minimum(x, 0.0)) - 1.0)


def _alpha_block(h, a_src_ref, a_dst_ref, i, nblk, asrc_ref, adst_ref,
                 shift_ref, mx_ref):
    av = jnp.dot(h, a_src_ref[...], preferred_element_type=jnp.float32)
    bv = jnp.dot(h, a_dst_ref[...], preferred_element_type=jnp.float32)
    asrc_ref[...] = av
    adst_ref[...] = bv
    am = jnp.max(av)
    bm = jnp.max(bv)

    @pl.when(i == 0)
    def _():
        mx_ref[0] = am
        mx_ref[1] = bm

    @pl.when(i > 0)
    def _():
        mx_ref[0] = jnp.maximum(mx_ref[0], am)
        mx_ref[1] = jnp.maximum(mx_ref[1], bm)

    @pl.when(i == nblk - 1)
    def _():
        shift_ref[...] = jnp.full(
            (1, 1),
            jnp.maximum(mx_ref[0], 0.0) + jnp.maximum(mx_ref[1], 0.0),
            jnp.float32)


def _tc1_body(x_ref, w_ref, a_src_ref, a_dst_ref,
              h_ref, asrc_ref, adst_ref, shift_ref, mx_ref):
    h = jnp.dot(x_ref[...], w_ref[...], preferred_element_type=jnp.float32)
    h_ref[...] = h
    _alpha_block(h, a_src_ref, a_dst_ref, pl.program_id(0), pl.num_programs(0),
                 asrc_ref, adst_ref, shift_ref, mx_ref)


def _tc1(x, W, a_src, a_dst):
    nblk = NPAD // BN
    return pl.pallas_call(
        _tc1_body,
        grid=(nblk,),
        in_specs=[
            pl.BlockSpec((BN, D), lambda i: (i, 0)),
            pl.BlockSpec((D, D), lambda i: (0, 0)),
            pl.BlockSpec((D, 1), lambda i: (0, 0)),
            pl.BlockSpec((D, 1), lambda i: (0, 0)),
        ],
        out_specs=[
            pl.BlockSpec((BN, D), lambda i: (i, 0)),
            pl.BlockSpec((BN, 1), lambda i: (i, 0)),
            pl.BlockSpec((BN, 1), lambda i: (i, 0)),
            pl.BlockSpec((1, 1), lambda i: (0, 0)),
        ],
        out_shape=[
            jax.ShapeDtypeStruct((NPAD, D), jnp.float32),
            jax.ShapeDtypeStruct((NPAD, 1), jnp.float32),
            jax.ShapeDtypeStruct((NPAD, 1), jnp.float32),
            jax.ShapeDtypeStruct((1, 1), jnp.float32),
        ],
        scratch_shapes=[pltpu.SMEM((2,), jnp.float32)],
    )(x, W, a_src, a_dst)


def _tc2_body(agg_ref, den_ref, b_ref, g_ref, lb_ref, w_ref,
              a_src_ref, a_dst_ref,
              h_ref, asrc_ref, adst_ref, shift_ref, mx_ref):
    o = agg_ref[...] / (den_ref[...] + 1e-16) + b_ref[...]
    o = _elu(o)
    mu = jnp.mean(o, axis=-1, keepdims=True)
    var = jnp.mean((o - mu) ** 2, axis=-1, keepdims=True)
    hn = (o - mu) / jnp.sqrt(var + 1e-5) * g_ref[...] + lb_ref[...]
    h = jnp.dot(hn, w_ref[...], preferred_element_type=jnp.float32)
    h_ref[...] = h
    _alpha_block(h, a_src_ref, a_dst_ref, pl.program_id(0), pl.num_programs(0),
                 asrc_ref, adst_ref, shift_ref, mx_ref)


def _tc2(agg, den, b, g, lb, W, a_src, a_dst):
    nblk = NPAD // BN
    return pl.pallas_call(
        _tc2_body,
        grid=(nblk,),
        in_specs=[
            pl.BlockSpec((BN, D), lambda i: (i, 0)),
            pl.BlockSpec((BN, 1), lambda i: (i, 0)),
            pl.BlockSpec((1, D), lambda i: (0, 0)),
            pl.BlockSpec((1, D), lambda i: (0, 0)),
            pl.BlockSpec((1, D), lambda i: (0, 0)),
            pl.BlockSpec((D, D), lambda i: (0, 0)),
            pl.BlockSpec((D, 1), lambda i: (0, 0)),
            pl.BlockSpec((D, 1), lambda i: (0, 0)),
        ],
        out_specs=[
            pl.BlockSpec((BN, D), lambda i: (i, 0)),
            pl.BlockSpec((BN, 1), lambda i: (i, 0)),
            pl.BlockSpec((BN, 1), lambda i: (i, 0)),
            pl.BlockSpec((1, 1), lambda i: (0, 0)),
        ],
        out_shape=[
            jax.ShapeDtypeStruct((NPAD, D), jnp.float32),
            jax.ShapeDtypeStruct((NPAD, 1), jnp.float32),
            jax.ShapeDtypeStruct((NPAD, 1), jnp.float32),
            jax.ShapeDtypeStruct((1, 1), jnp.float32),
        ],
        scratch_shapes=[pltpu.SMEM((2,), jnp.float32)],
    )(agg, den, b, g, lb, W, a_src, a_dst)


def _tc3_body(agg_ref, den_ref, b_ref, ew1_ref, eb1_ref, ew2_ref, eb2_ref,
              pw_ref, pb_ref, x_ref, tw_ref, tb_ref, zp_ref, zt_ref):
    o = agg_ref[...] / (den_ref[...] + 1e-16) + b_ref[...]
    o = _elu(o)
    he = jnp.maximum(
        jnp.dot(o, ew1_ref[...], preferred_element_type=jnp.float32)
        + eb1_ref[...], 0.0)
    he = jnp.dot(he, ew2_ref[...], preferred_element_type=jnp.float32) \
        + eb2_ref[...]
    zp_ref[...] = jnp.dot(he, pw_ref[...],
                          preferred_element_type=jnp.float32) + pb_ref[...]
    zt_ref[...] = jnp.dot(x_ref[...], tw_ref[...],
                          preferred_element_type=jnp.float32) + tb_ref[...]


def _tc3(agg, den, b, ew1, eb1, ew2, eb2, pw, pb, x, tw, tb):
    nblk = NPAD // BN
    full = lambda r, c: pl.BlockSpec((r, c), lambda i: (0, 0))
    blk = lambda c: pl.BlockSpec((BN, c), lambda i: (i, 0))
    return pl.pallas_call(
        _tc3_body,
        grid=(nblk,),
        in_specs=[
            blk(D), pl.BlockSpec((BN, 1), lambda i: (i, 0)), full(1, D),
            full(D, D), full(1, D), full(D, D), full(1, D),
            full(D, LAT), full(1, LAT),
            blk(D), full(D, LAT), full(1, LAT),
        ],
        out_specs=[blk(LAT), blk(LAT)],
        out_shape=[
            jax.ShapeDtypeStruct((NPAD, LAT), jnp.float32),
            jax.ShapeDtypeStruct((NPAD, LAT), jnp.float32),
        ],
    )(agg, den, b, ew1, eb1, ew2, eb2, pw, pb, x, tw, tb)


def _sc_ex_body(src_hbm, dst_hbm, asrc_hbm, adst_hbm, shift_hbm,
                ex_hbm, src_v, dst_v, av, bv, shift_v, sem):
    c = lax.axis_index("c")
    s = lax.axis_index("s")
    w = s * NC + c
    base = w * EW1

    pltpu.sync_copy(shift_hbm, shift_v)
    pltpu.sync_copy(src_hbm.at[pl.ds(base, EW1)], src_v)
    pltpu.sync_copy(dst_hbm.at[pl.ds(base, EW1)], dst_v)

    def _gather(i, carry):
        d1 = pltpu.async_copy(asrc_hbm.at[src_v.at[pl.ds(i * G, G)]],
                              av.at[pl.ds(i * G, G)], sem)
        d2 = pltpu.async_copy(adst_hbm.at[dst_v.at[pl.ds(i * G, G)]],
                              bv.at[pl.ds(i * G, G)], sem)
        d1.wait()
        d2.wait()
        return carry

    lax.fori_loop(0, EW1 // G, _gather, 0)

    shift = shift_v[...]

    def _ex(i, carry):
        a = av[pl.ds(i * L, L)] + bv[pl.ds(i * L, L)]
        e = jnp.where(a >= 0, a, NEG * a)
        av[pl.ds(i * L, L)] = jnp.exp(e - shift)
        return carry

    lax.fori_loop(0, EW1 // L, _ex, 0)
    pltpu.sync_copy(av, ex_hbm.at[pl.ds(base, EW1)])


@functools.cache
def _sc_ex_kernel():
    return pl.kernel(
        _sc_ex_body,
        compiler_params=_SC_PARAMS,
        out_type=[jax.ShapeDtypeStruct((EPAD,), jnp.float32)],
        mesh=plsc.VectorSubcoreMesh(core_axis_name="c", subcore_axis_name="s",
                                    num_cores=NC, num_subcores=NS),
        scratch_types=[
            pltpu.VMEM((EW1,), jnp.int32),
            pltpu.VMEM((EW1,), jnp.int32),
            pltpu.VMEM((EW1,), jnp.float32),
            pltpu.VMEM((EW1,), jnp.float32),
            pltpu.VMEM((L,), jnp.float32),
            pltpu.SemaphoreType.DMA,
        ],
    )


def _sc_agg_body(src_hbm, dst_hbm, ex_hbm, h_hbm,
                 agg_hbm, den_hbm,
                 srcc, dstc, exc, crow, csrc, cex, gsrc, grow, gex,
                 rows, acc, denloc, sem0, sem1, gsem):
    c = lax.axis_index("c")
    s = lax.axis_index("s")
    w = s * NC + c
    lo = w * RNG

    zf = jnp.zeros((L,), jnp.float32)
    zi = jnp.zeros((L,), jnp.int32)
    dummy = jnp.full((L,), RNG, jnp.int32)

    def _zacc(i, carry):
        r = i // (D // L)
        col = (i % (D // L)) * L
        acc[r, pl.ds(col, L)] = zf
        return carry

    lax.fori_loop(0, (RNG + 8) * (D // L), _zacc, 0)

    def _zden(i, carry):
        denloc[pl.ds(i * L, L)] = zf
        return carry

    lax.fori_loop(0, (RNG + 16) // L, _zden, 0)

    def _accum_from(rowsrc_ref, exsrc_ref, off):
        def _acc16(r16, carry2):
            cb = off + r16 * L
            rb = r16 * L
            exw = exsrc_ref[pl.ds(cb, L)]
            rv = rowsrc_ref[pl.ds(cb, L)]
            plsc.addupdate_scatter(denloc, [rv], exw)
            for k in range(L):
                wv = jnp.full((L,), exw[k], jnp.float32)
                row = rv[k]
                vals = [rows[rb + k, pl.ds(j * L, L)] * wv
                        for j in range(D // L)]
                for j in range(D // L):
                    plsc.addupdate(acc.at[row, pl.ds(j * L, L)], vals[j])
            return carry2

        lax.fori_loop(0, G // L, _acc16, 0)

    def _retire(pend):
        @pl.when(pend == 1)
        def _():
            pltpu.make_async_copy(h_hbm.at[gsrc], rows, gsem).wait()
            _accum_from(grow, gex, 0)

    def _start_chunk(ch, b, sem):
        base = ch * C
        pltpu.async_copy(src_hbm.at[pl.ds(base, C)], srcc.at[b], sem)
        pltpu.async_copy(dst_hbm.at[pl.ds(base, C)], dstc.at[b], sem)
        pltpu.async_copy(ex_hbm.at[pl.ds(base, C)], exc.at[b], sem)

    def _wait_chunk(ch, b, sem):
        base = ch * C
        pltpu.make_async_copy(src_hbm.at[pl.ds(base, C)], srcc.at[b],
                              sem).wait()
        pltpu.make_async_copy(dst_hbm.at[pl.ds(base, C)], dstc.at[b],
                              sem).wait()
        pltpu.make_async_copy(ex_hbm.at[pl.ds(base, C)], exc.at[b],
                              sem).wait()

    def _scan_chunk(b, cnt):
        def _scan(i, cn):
            for u in range(4):
                ii = i * 4 + u
                d = dstc[b, pl.ds(ii * L, L)]
                m = (d >= lo) & (d < lo + RNG)
                plsc.store_compressed(crow.at[pl.ds(cn, L)], d - lo, mask=m)
                plsc.store_compressed(csrc.at[pl.ds(cn, L)],
                                      srcc[b, pl.ds(ii * L, L)], mask=m)
                plsc.store_compressed(cex.at[pl.ds(cn, L)],
                                      exc[b, pl.ds(ii * L, L)], mask=m)
                cn = cn + plsc.all_reduce_population_count(m)[0]
            return cn

        return lax.fori_loop(0, C // L // 4, _scan, cnt)

    def _boundary(cnt, pend):
        nd = cnt // G

        @pl.when(nd >= 1)
        def _():
            _retire(pend)

            def _extra(g, carry):
                pltpu.async_copy(h_hbm.at[csrc.at[pl.ds(g * G, G)]], rows,
                                 gsem).wait()
                _accum_from(crow, cex, g * G)
                return carry

            lax.fori_loop(1, jnp.maximum(nd, 1), _extra, 0)

            for t in range(G // L):
                gsrc[pl.ds(t * L, L)] = csrc[pl.ds(t * L, L)]
                grow[pl.ds(t * L, L)] = crow[pl.ds(t * L, L)]
                gex[pl.ds(t * L, L)] = cex[pl.ds(t * L, L)]
            pltpu.async_copy(h_hbm.at[gsrc], rows, gsem)

            for t in range(G // L):
                crow[pl.ds(t * L, L)] = crow[pl.ds(nd * G + t * L, L)]
                csrc[pl.ds(t * L, L)] = csrc[pl.ds(nd * G + t * L, L)]
                cex[pl.ds(t * L, L)] = cex[pl.ds(nd * G + t * L, L)]

        pend_new = jnp.where(nd >= 1, jnp.int32(1), pend)
        return cnt - nd * G, pend_new

    _start_chunk(0, 0, sem0)

    def _pair(p, state):
        cnt, pend = state
        ch0 = 2 * p
        _start_chunk(ch0 + 1, 1, sem1)
        _wait_chunk(ch0, 0, sem0)
        cnt = _scan_chunk(0, cnt)
        cnt, pend = _boundary(cnt, pend)

        @pl.when(ch0 + 2 < NCH)
        def _():
            _start_chunk(ch0 + 2, 0, sem0)

        _wait_chunk(ch0 + 1, 1, sem1)
        cnt = _scan_chunk(1, cnt)
        return _boundary(cnt, pend)

    cnt, pend = lax.fori_loop(0, NCH // 2, _pair,
                              (jnp.int32(0), jnp.int32(0)))
    _retire(pend)

    for t in range(G // L):
        crow[pl.ds(cnt + t * L, L)] = dummy
        csrc[pl.ds(cnt + t * L, L)] = zi
        cex[pl.ds(cnt + t * L, L)] = zf

    @pl.when(cnt > 0)
    def _():
        pltpu.async_copy(h_hbm.at[csrc.at[pl.ds(0, G)]], rows, gsem).wait()
        _accum_from(crow, cex, 0)

    pltpu.sync_copy(acc.at[pl.ds(0, RNG)], agg_hbm.at[pl.ds(lo, RNG)])
    pltpu.sync_copy(denloc.at[pl.ds(0, RNG)], den_hbm.at[pl.ds(lo, RNG)])


@functools.cache
def _sc_agg_kernel():
    return pl.kernel(
        _sc_agg_body,
        compiler_params=_SC_PARAMS,
        out_type=[
            jax.ShapeDtypeStruct((NPAD, D), jnp.float32),
            jax.ShapeDtypeStruct((NPAD,), jnp.float32),
        ],
        mesh=plsc.VectorSubcoreMesh(core_axis_name="c", subcore_axis_name="s",
                                    num_cores=NC, num_subcores=NS),
        scratch_types=[
            pltpu.VMEM((2, C), jnp.int32),
            pltpu.VMEM((2, C), jnp.int32),
            pltpu.VMEM((2, C), jnp.float32),
            pltpu.VMEM((CAP,), jnp.int32),
            pltpu.VMEM((CAP,), jnp.int32),
            pltpu.VMEM((CAP,), jnp.float32),
            pltpu.VMEM((G,), jnp.int32),
            pltpu.VMEM((G,), jnp.int32),
            pltpu.VMEM((G,), jnp.float32),
            pltpu.VMEM((G, D), jnp.float32),
            pltpu.VMEM((RNG + 8, D), jnp.float32),
            pltpu.VMEM((RNG + 16,), jnp.float32),
            pltpu.SemaphoreType.DMA,
            pltpu.SemaphoreType.DMA,
            pltpu.SemaphoreType.DMA,
        ],
    )


def kernel(x, edge_index, W0, a_src0, a_dst0, b0, ln0_g, ln0_b, W1, a_src1,
           a_dst1, b1, encW1, encb1, encW2, encb2, predW, predb, tgtW, tgtb):
    src = edge_index[0].astype(jnp.int32)
    dst = edge_index[1].astype(jnp.int32)
    src_p = jnp.concatenate([src, jnp.zeros((EPAD - E,), jnp.int32)])
    dst_p = jnp.concatenate([dst, jnp.full((EPAD - E,), N, jnp.int32)])
    x_pad = jnp.zeros((NPAD, D), jnp.float32).at[:N].set(x)

    h0, asrc0, adst0, shift0 = _tc1(x_pad, W0, a_src0.reshape(D, 1),
                                    a_dst0.reshape(D, 1))
    sv0 = jnp.broadcast_to(shift0.reshape(()), (L,))
    ex0, = _sc_ex_kernel()(src_p, dst_p, asrc0.reshape(-1), adst0.reshape(-1),
                           sv0)
    agg0, den0 = _sc_agg_kernel()(src_p, dst_p, ex0, h0)

    h1, asrc1, adst1, shift1 = _tc2(agg0, den0.reshape(NPAD, 1),
                                    b0.reshape(1, D), ln0_g.reshape(1, D),
                                    ln0_b.reshape(1, D), W1,
                                    a_src1.reshape(D, 1), a_dst1.reshape(D, 1))
    sv1 = jnp.broadcast_to(shift1.reshape(()), (L,))
    ex1, = _sc_ex_kernel()(src_p, dst_p, asrc1.reshape(-1), adst1.reshape(-1),
                           sv1)
    agg1, den1 = _sc_agg_kernel()(src_p, dst_p, ex1, h1)

    zp, zt = _tc3(agg1, den1.reshape(NPAD, 1), b1.reshape(1, D),
                  encW1, encb1.reshape(1, D), encW2, encb2.reshape(1, D),
                  predW, predb.reshape(1, LAT), x_pad, tgtW,
                  tgtb.reshape(1, LAT))
    return (zp[:N], zt[:N])
